# baseline (speedup 1.0000x reference)
import functools

import jax
import jax.numpy as jnp
from jax import lax
from jax.experimental import pallas as pl
from jax.experimental.pallas import tpu as pltpu
from jax.experimental.pallas import tpu_sc as plsc

N = 10000
E = 640000
D_FEAT = 128
HIDDEN = 32
LAYERS = 3
NUM_GRAPHS = 64
BN_EPS = 1e-5

NW = 32
EPB = 128
ROWS_W = 160
E_PAD = NW * ROWS_W * EPB
N_PAD = 10240
STRIPE = N_PAD // 16


_GRP = 8
_HALF = ROWS_W // (2 * _GRP)


def _sc_pass(h_hbm, zeros_hbm, out_slot, idx_s, idx_d, rows, acc, sem, s):
    for b in range(_GRP):
        pltpu.async_copy(h_hbm.at[idx_s.at[b]], rows.at[b], sem)
    for b in range(STRIPE // EPB):
        pltpu.sync_copy(zeros_hbm,
                        acc.at[pl.ds(s * STRIPE + b * EPB, EPB)])
    plsc.subcore_barrier()

    def body(i, carry):
        ja = i * (2 * _GRP)
        jb = ja + _GRP
        jn = lax.rem(ja + 2 * _GRP, ROWS_W)
        for b in range(_GRP):
            pltpu.make_async_copy(h_hbm.at[idx_s.at[ja + b]], rows.at[b],
                                  sem).wait()
        for b in range(_GRP):
            pltpu.async_copy(h_hbm.at[idx_s.at[jb + b]], rows.at[_GRP + b],
                             sem)
        for b in range(_GRP):
            pltpu.sync_copy(rows.at[b], acc.at[idx_d.at[ja + b]], add=True)
        for b in range(_GRP):
            pltpu.make_async_copy(h_hbm.at[idx_s.at[jb + b]],
                                  rows.at[_GRP + b], sem).wait()
        @pl.when(i < _HALF - 1)
        def _fire_next():
            for b in range(_GRP):
                pltpu.async_copy(h_hbm.at[idx_s.at[jn + b]], rows.at[b], sem)
        for b in range(_GRP):
            pltpu.sync_copy(rows.at[_GRP + b], acc.at[idx_d.at[jb + b]],
                            add=True)
        return carry

    lax.fori_loop(0, _HALF, body, 0)
    plsc.subcore_barrier()
    pltpu.sync_copy(acc.at[pl.ds(s * STRIPE, STRIPE)],
                    out_slot.at[pl.ds(s * STRIPE, STRIPE)])


def _sc_scatter_body(src_hbm, dst_hbm, h_hbm, zeros_hbm, out_hbm,
                     idx_s, idx_d, rows, acc, sem):
    c = lax.axis_index("c")
    s = lax.axis_index("s")
    wid = c * 16 + s
    pltpu.sync_copy(src_hbm.at[wid], idx_s)
    pltpu.sync_copy(dst_hbm.at[wid], idx_d)
    _sc_pass(h_hbm, zeros_hbm, out_hbm.at[c], idx_s, idx_d, rows, acc, sem, s)


def _sc_scatter4_body(src_hbm, dst_hbm, h0, h1, h2, h3, zeros_hbm, out_hbm,
                      idx_s, idx_d, rows, acc, sem):
    c = lax.axis_index("c")
    s = lax.axis_index("s")
    wid = c * 16 + s
    pltpu.sync_copy(src_hbm.at[wid], idx_s)
    pltpu.sync_copy(dst_hbm.at[wid], idx_d)
    for k, h_hbm in enumerate((h0, h1, h2, h3)):
        _sc_pass(h_hbm, zeros_hbm, out_hbm.at[k].at[c], idx_s, idx_d, rows,
                 acc, sem, s)


def _sc_scratch(width):
    return [
        pltpu.VMEM((ROWS_W, EPB), jnp.int32),
        pltpu.VMEM((ROWS_W, EPB), jnp.int32),
        pltpu.VMEM((2 * _GRP, EPB, width), jnp.float32),
        pltpu.VMEM_SHARED((N_PAD, width), jnp.float32),
        pltpu.SemaphoreType.DMA,
    ]


@functools.cache
def _sc_scatter_kernel(width):
    return pl.kernel(
        _sc_scatter_body,
        out_type=jax.ShapeDtypeStruct((2, N_PAD, width), jnp.float32),
        mesh=plsc.VectorSubcoreMesh(core_axis_name="c", subcore_axis_name="s"),
        scratch_types=_sc_scratch(width),
        compiler_params=pltpu.CompilerParams(use_tc_tiling_on_sc=False),
    )


@functools.cache
def _sc_scatter4_kernel():
    return pl.kernel(
        _sc_scatter4_body,
        out_type=jax.ShapeDtypeStruct((4, 2, N_PAD, HIDDEN), jnp.float32),
        mesh=plsc.VectorSubcoreMesh(core_axis_name="c", subcore_axis_name="s"),
        scratch_types=_sc_scratch(HIDDEN),
        compiler_params=pltpu.CompilerParams(use_tc_tiling_on_sc=False),
    )


def _sc_scatter(src, dst, h):
    width = h.shape[-1]
    zeros = jnp.zeros((EPB, width), jnp.float32)
    return _sc_scatter_kernel(width)(src, dst, h, zeros)


def _gin_layer(h, p0, p1, w1, b1, w2, b2, g, be):
    t = jnp.dot(h + p0 + p1, w1, preferred_element_type=jnp.float32) + b1
    t = jnp.maximum(t, 0.0)
    t = jnp.dot(t, w2, preferred_element_type=jnp.float32) + b2
    t = jnp.maximum(t, 0.0)
    mask = lax.broadcasted_iota(jnp.int32, (N_PAD, 1), 0) < N
    mean = jnp.sum(jnp.where(mask, t, 0.0), axis=0, keepdims=True) / N
    var = jnp.sum(jnp.where(mask, (t - mean) ** 2, 0.0), axis=0,
                  keepdims=True) / N
    z = (t - mean) / jnp.sqrt(var + BN_EPS) * g + be
    return jnp.where(mask, z, 0.0)


def _mid_body(h_ref, p_ref, w1_ref, b1_ref, w2_ref, b2_ref, g_ref, be_ref,
              z_ref):
    z_ref[...] = _gin_layer(h_ref[...], p_ref[0], p_ref[1], w1_ref[...],
                            b1_ref[...], w2_ref[...], b2_ref[...],
                            g_ref[...], be_ref[...])


@functools.cache
def _mid_kernel():
    return pl.pallas_call(
        _mid_body,
        out_shape=jax.ShapeDtypeStruct((N_PAD, HIDDEN), jnp.float32),
    )


def _mid0_body(h_ref, p_ref, w1_ref, b1_ref, w2_ref, b2_ref, g_ref, be_ref,
               z_ref):
    agg = jnp.concatenate([p_ref[k, 0] + p_ref[k, 1] for k in range(4)],
                          axis=1)
    z_ref[...] = _gin_layer(h_ref[...], agg, 0.0, w1_ref[...], b1_ref[...],
                            w2_ref[...], b2_ref[...], g_ref[...], be_ref[...])


@functools.cache
def _mid0_kernel():
    return pl.pallas_call(
        _mid0_body,
        out_shape=jax.ShapeDtypeStruct((N_PAD, HIDDEN), jnp.float32),
    )


def _last_body(h_ref, p_ref, w1_ref, b1_ref, w2_ref, b2_ref, g_ref, be_ref,
               z0_ref, z1_ref, batch_ref, o_ref):
    z2 = _gin_layer(h_ref[...], p_ref[0], p_ref[1], w1_ref[...], b1_ref[...],
                    w2_ref[...], b2_ref[...], g_ref[...], be_ref[...])
    oh_t = (lax.broadcasted_iota(jnp.int32, (NUM_GRAPHS, N_PAD), 0)
            == batch_ref[...]).astype(jnp.float32)
    hp = lax.Precision.HIGHEST
    p0 = jnp.dot(oh_t, z0_ref[...], preferred_element_type=jnp.float32,
                 precision=hp)
    p1 = jnp.dot(oh_t, z1_ref[...], preferred_element_type=jnp.float32,
                 precision=hp)
    p2 = jnp.dot(oh_t, z2, preferred_element_type=jnp.float32, precision=hp)
    o_ref[...] = jnp.concatenate([p0, p1, p2], axis=1)


@functools.cache
def _last_kernel():
    return pl.pallas_call(
        _last_body,
        out_shape=jax.ShapeDtypeStruct((NUM_GRAPHS, LAYERS * HIDDEN),
                                       jnp.float32),
    )


def kernel(x, edge_index, batch, params):
    pad = N + jnp.arange(E_PAD - E, dtype=jnp.int32) % (N_PAD - N)
    src = jnp.concatenate([edge_index[0].astype(jnp.int32), pad])
    dst = jnp.concatenate([edge_index[1].astype(jnp.int32), pad])
    src = src.reshape(NW, ROWS_W, EPB)
    dst = dst.reshape(NW, ROWS_W, EPB)
    batch_p = jnp.concatenate(
        [batch.astype(jnp.int32),
         jnp.full((N_PAD - N,), NUM_GRAPHS, jnp.int32)]).reshape(1, N_PAD)
    x_pad = jnp.pad(x, ((0, N_PAD - N), (0, 0)))

    def lp(name, i):
        v = params[name + '_%d' % i]
        return v.reshape(1, -1) if v.ndim == 1 else v

    def layer_params(i):
        return (params['W1_%d' % i], lp('b1', i), params['W2_%d' % i],
                lp('b2', i), lp('gamma', i), lp('beta', i))

    zeros = jnp.zeros((EPB, HIDDEN), jnp.float32)
    a0 = _sc_scatter4_kernel()(src, dst,
                               *[x_pad[:, 32 * k:32 * (k + 1)]
                                 for k in range(4)], zeros)
    z0 = _mid0_kernel()(x_pad, a0, *layer_params(0))
    a1 = _sc_scatter(src, dst, z0)
    z1 = _mid_kernel()(z0, a1, *layer_params(1))
    a2 = _sc_scatter(src, dst, z1)
    out = _last_kernel()(z1, a2, *layer_params(2), z0, z1, batch_p)
    return out


# speedup vs baseline: 1.0261x; 1.0261x over previous
import functools

import jax
import jax.numpy as jnp
from jax import lax
from jax.experimental import pallas as pl
from jax.experimental.pallas import tpu as pltpu
from jax.experimental.pallas import tpu_sc as plsc

N = 10000
E = 640000
D_FEAT = 128
HIDDEN = 32
LAYERS = 3
NUM_GRAPHS = 64
BN_EPS = 1e-5

NW = 32
EPB = 128
ROWS_W = 160
E_PAD = NW * ROWS_W * EPB
N_PAD = 10240
STRIPE = N_PAD // 16


_GRP = 8
_HALF = ROWS_W // (2 * _GRP)


def _sc_pass(h_hbm, zeros_hbm, out_slot, idx_s, idx_d, rows, acc, sem,
             sem2, s):
    for b in range(_GRP):
        pltpu.async_copy(h_hbm.at[idx_s.at[b]], rows.at[b], sem)
    for b in range(STRIPE // EPB):
        pltpu.async_copy(zeros_hbm,
                         acc.at[pl.ds(s * STRIPE + b * EPB, EPB)], sem2)
    for b in range(STRIPE // EPB):
        pltpu.make_async_copy(zeros_hbm,
                              acc.at[pl.ds(s * STRIPE + b * EPB, EPB)],
                              sem2).wait()
    plsc.subcore_barrier()

    def body(i, carry):
        ja = i * (2 * _GRP)
        jb = ja + _GRP
        jn = lax.rem(ja + 2 * _GRP, ROWS_W)
        for b in range(_GRP):
            pltpu.make_async_copy(h_hbm.at[idx_s.at[ja + b]], rows.at[b],
                                  sem).wait()
        for b in range(_GRP):
            pltpu.async_copy(h_hbm.at[idx_s.at[jb + b]], rows.at[_GRP + b],
                             sem)
        for b in range(_GRP):
            pltpu.sync_copy(rows.at[b], acc.at[idx_d.at[ja + b]], add=True)
        for b in range(_GRP):
            pltpu.make_async_copy(h_hbm.at[idx_s.at[jb + b]],
                                  rows.at[_GRP + b], sem).wait()
        @pl.when(i < _HALF - 1)
        def _fire_next():
            for b in range(_GRP):
                pltpu.async_copy(h_hbm.at[idx_s.at[jn + b]], rows.at[b], sem)
        for b in range(_GRP):
            pltpu.sync_copy(rows.at[_GRP + b], acc.at[idx_d.at[jb + b]],
                            add=True)
        return carry

    lax.fori_loop(0, _HALF, body, 0)
    plsc.subcore_barrier()
    pltpu.sync_copy(acc.at[pl.ds(s * STRIPE, STRIPE)],
                    out_slot.at[pl.ds(s * STRIPE, STRIPE)])


def _sc_scatter_body(src_hbm, dst_hbm, h_hbm, zeros_hbm, out_hbm,
                     idx_s, idx_d, rows, acc, sem, sem2):
    c = lax.axis_index("c")
    s = lax.axis_index("s")
    wid = c * 16 + s
    pltpu.sync_copy(src_hbm.at[wid], idx_s)
    pltpu.sync_copy(dst_hbm.at[wid], idx_d)
    _sc_pass(h_hbm, zeros_hbm, out_hbm.at[c], idx_s, idx_d, rows, acc, sem,
             sem2, s)


def _sc_scatter4_body(src_hbm, dst_hbm, h0, h1, h2, h3, zeros_hbm, out_hbm,
                      idx_s, idx_d, rows, acc, sem, sem2):
    c = lax.axis_index("c")
    s = lax.axis_index("s")
    wid = c * 16 + s
    pltpu.sync_copy(src_hbm.at[wid], idx_s)
    pltpu.sync_copy(dst_hbm.at[wid], idx_d)
    for k, h_hbm in enumerate((h0, h1, h2, h3)):
        _sc_pass(h_hbm, zeros_hbm, out_hbm.at[k].at[c], idx_s, idx_d, rows,
                 acc, sem, sem2, s)


def _sc_scratch(width):
    return [
        pltpu.VMEM((ROWS_W, EPB), jnp.int32),
        pltpu.VMEM((ROWS_W, EPB), jnp.int32),
        pltpu.VMEM((2 * _GRP, EPB, width), jnp.float32),
        pltpu.VMEM_SHARED((N_PAD, width), jnp.float32),
        pltpu.SemaphoreType.DMA,
        pltpu.SemaphoreType.DMA,
    ]


@functools.cache
def _sc_scatter_kernel(width):
    return pl.kernel(
        _sc_scatter_body,
        out_type=jax.ShapeDtypeStruct((2, N_PAD, width), jnp.float32),
        mesh=plsc.VectorSubcoreMesh(core_axis_name="c", subcore_axis_name="s"),
        scratch_types=_sc_scratch(width),
        compiler_params=pltpu.CompilerParams(use_tc_tiling_on_sc=False),
    )


@functools.cache
def _sc_scatter4_kernel():
    return pl.kernel(
        _sc_scatter4_body,
        out_type=jax.ShapeDtypeStruct((4, 2, N_PAD, HIDDEN), jnp.float32),
        mesh=plsc.VectorSubcoreMesh(core_axis_name="c", subcore_axis_name="s"),
        scratch_types=_sc_scratch(HIDDEN),
        compiler_params=pltpu.CompilerParams(use_tc_tiling_on_sc=False),
    )


def _sc_scatter(src, dst, h):
    width = h.shape[-1]
    zeros = jnp.zeros((EPB, width), jnp.float32)
    return _sc_scatter_kernel(width)(src, dst, h, zeros)


def _gin_layer(h, p0, p1, w1, b1, w2, b2, g, be):
    t = jnp.dot(h + p0 + p1, w1, preferred_element_type=jnp.float32) + b1
    t = jnp.maximum(t, 0.0)
    t = jnp.dot(t, w2, preferred_element_type=jnp.float32) + b2
    t = jnp.maximum(t, 0.0)
    mask = lax.broadcasted_iota(jnp.int32, (N_PAD, 1), 0) < N
    mean = jnp.sum(jnp.where(mask, t, 0.0), axis=0, keepdims=True) / N
    var = jnp.sum(jnp.where(mask, (t - mean) ** 2, 0.0), axis=0,
                  keepdims=True) / N
    z = (t - mean) / jnp.sqrt(var + BN_EPS) * g + be
    return jnp.where(mask, z, 0.0)


def _mid_body(h_ref, p_ref, w1_ref, b1_ref, w2_ref, b2_ref, g_ref, be_ref,
              z_ref):
    z_ref[...] = _gin_layer(h_ref[...], p_ref[0], p_ref[1], w1_ref[...],
                            b1_ref[...], w2_ref[...], b2_ref[...],
                            g_ref[...], be_ref[...])


@functools.cache
def _mid_kernel():
    return pl.pallas_call(
        _mid_body,
        out_shape=jax.ShapeDtypeStruct((N_PAD, HIDDEN), jnp.float32),
    )


def _mid0_body(h_ref, p_ref, w1_ref, b1_ref, w2_ref, b2_ref, g_ref, be_ref,
               z_ref):
    agg = jnp.concatenate([p_ref[k, 0] + p_ref[k, 1] for k in range(4)],
                          axis=1)
    z_ref[...] = _gin_layer(h_ref[...], agg, 0.0, w1_ref[...], b1_ref[...],
                            w2_ref[...], b2_ref[...], g_ref[...], be_ref[...])


@functools.cache
def _mid0_kernel():
    return pl.pallas_call(
        _mid0_body,
        out_shape=jax.ShapeDtypeStruct((N_PAD, HIDDEN), jnp.float32),
    )


def _last_body(h_ref, p_ref, w1_ref, b1_ref, w2_ref, b2_ref, g_ref, be_ref,
               z0_ref, z1_ref, batch_ref, o_ref):
    z2 = _gin_layer(h_ref[...], p_ref[0], p_ref[1], w1_ref[...], b1_ref[...],
                    w2_ref[...], b2_ref[...], g_ref[...], be_ref[...])
    oh_t = (lax.broadcasted_iota(jnp.int32, (NUM_GRAPHS, N_PAD), 0)
            == batch_ref[...]).astype(jnp.float32)
    hp = lax.Precision.HIGHEST
    p0 = jnp.dot(oh_t, z0_ref[...], preferred_element_type=jnp.float32,
                 precision=hp)
    p1 = jnp.dot(oh_t, z1_ref[...], preferred_element_type=jnp.float32,
                 precision=hp)
    p2 = jnp.dot(oh_t, z2, preferred_element_type=jnp.float32, precision=hp)
    o_ref[...] = jnp.concatenate([p0, p1, p2], axis=1)


@functools.cache
def _last_kernel():
    return pl.pallas_call(
        _last_body,
        out_shape=jax.ShapeDtypeStruct((NUM_GRAPHS, LAYERS * HIDDEN),
                                       jnp.float32),
    )


def kernel(x, edge_index, batch, params):
    pad = N + jnp.arange(E_PAD - E, dtype=jnp.int32) % (N_PAD - N)
    src = jnp.concatenate([edge_index[0].astype(jnp.int32), pad])
    dst = jnp.concatenate([edge_index[1].astype(jnp.int32), pad])
    src = src.reshape(NW, ROWS_W, EPB)
    dst = dst.reshape(NW, ROWS_W, EPB)
    batch_p = jnp.concatenate(
        [batch.astype(jnp.int32),
         jnp.full((N_PAD - N,), NUM_GRAPHS, jnp.int32)]).reshape(1, N_PAD)
    x_pad = jnp.pad(x, ((0, N_PAD - N), (0, 0)))

    def lp(name, i):
        v = params[name + '_%d' % i]
        return v.reshape(1, -1) if v.ndim == 1 else v

    def layer_params(i):
        return (params['W1_%d' % i], lp('b1', i), params['W2_%d' % i],
                lp('b2', i), lp('gamma', i), lp('beta', i))

    zeros = jnp.zeros((EPB, HIDDEN), jnp.float32)
    a0 = _sc_scatter4_kernel()(src, dst,
                               *[x_pad[:, 32 * k:32 * (k + 1)]
                                 for k in range(4)], zeros)
    z0 = _mid0_kernel()(x_pad, a0, *layer_params(0))
    a1 = _sc_scatter(src, dst, z0)
    z1 = _mid_kernel()(z0, a1, *layer_params(1))
    a2 = _sc_scatter(src, dst, z1)
    out = _last_kernel()(z1, a2, *layer_params(2), z0, z1, batch_p)
    return out
